# baseline (speedup 1.0000x reference)
import jax
import jax.numpy as jnp
from jax import lax
from jax.experimental import pallas as pl
from jax.experimental.pallas import tpu as pltpu
from jax.experimental.pallas import tpu_sc as plsc

F32 = jnp.float32
I32 = jnp.int32

NC = 2
NS = 16
NW = NC * NS

N = 10000
E = 320000
D = 128
B = 64
K = 30
EPW = E // NW
C = 200
NP = 10240
RPT = NP // NS
ZC = 64

_mesh = lambda: plsc.VectorSubcoreMesh(core_axis_name="c", subcore_axis_name="s")


def _zero_phase(zeros_hbm, zbuf, acc_sp, s):
    pltpu.sync_copy(zeros_hbm, zbuf)
    for k in range(RPT // ZC):
        pltpu.sync_copy(zbuf, acc_sp.at[pl.ds(s * RPT + k * ZC, ZC)])


def _writeback(out_hbm, zbuf, acc_sp, c, s):
    for k in range(RPT // ZC):
        pltpu.sync_copy(acc_sp.at[pl.ds(s * RPT + k * ZC, ZC)], zbuf)
        pltpu.sync_copy(zbuf, out_hbm.at[pl.ds(c * NP + s * RPT + k * ZC, ZC)])


CA = 80
NCH = EPW // CA


def _aggr_body(h_hbm, src_hbm, dst_hbm, zeros_hbm, out_hbm,
               sA, dA, sB, dB, zbuf, rowsA, rowsB, acc_sp, semA, semB):
    c = lax.axis_index("c")
    s = lax.axis_index("s")
    wid = s * NC + c
    _zero_phase(zeros_hbm, zbuf, acc_sp, s)
    plsc.subcore_barrier()

    base0 = wid * EPW
    pltpu.sync_copy(src_hbm.at[pl.ds(base0, CA)], sA)
    pltpu.sync_copy(dst_hbm.at[pl.ds(base0, CA)], dA)
    pltpu.async_copy(h_hbm.at[sA], rowsA, semA)

    def body(i, carry):
        ia = 2 * i
        ib = 2 * i + 1
        inext = 2 * i + 2
        pltpu.make_async_copy(h_hbm.at[sA], rowsA, semA).wait()

        @pl.when(ib < NCH)
        def _():
            base = wid * EPW + ib * CA
            pltpu.sync_copy(src_hbm.at[pl.ds(base, CA)], sB)
            pltpu.sync_copy(dst_hbm.at[pl.ds(base, CA)], dB)
            pltpu.async_copy(h_hbm.at[sB], rowsB, semB)

        pltpu.sync_copy(rowsA, acc_sp.at[dA], add=True)

        @pl.when(inext < NCH)
        def _():
            base = wid * EPW + inext * CA
            pltpu.sync_copy(src_hbm.at[pl.ds(base, CA)], sA)
            pltpu.sync_copy(dst_hbm.at[pl.ds(base, CA)], dA)
            pltpu.async_copy(h_hbm.at[sA], rowsA, semA)

        @pl.when(ib < NCH)
        def _():
            pltpu.make_async_copy(h_hbm.at[sB], rowsB, semB).wait()
            pltpu.sync_copy(rowsB, acc_sp.at[dB], add=True)
        return carry

    lax.fori_loop(0, (NCH + 1) // 2, body, 0)
    plsc.subcore_barrier()
    _writeback(out_hbm, zbuf, acc_sp, c, s)


def _aggr_call(h, src, dst):
    f = pl.kernel(
        _aggr_body,
        out_type=jax.ShapeDtypeStruct((NC * NP, D), F32),
        mesh=_mesh(),
        scratch_types=[
            pltpu.VMEM((CA,), I32),
            pltpu.VMEM((CA,), I32),
            pltpu.VMEM((CA,), I32),
            pltpu.VMEM((CA,), I32),
            pltpu.VMEM((ZC, D), F32),
            pltpu.VMEM((CA, D), F32),
            pltpu.VMEM((CA, D), F32),
            pltpu.VMEM_SHARED((NP, D), F32),
            pltpu.SemaphoreType.DMA,
            pltpu.SemaphoreType.DMA,
        ],
    )
    return f(h, src, dst, jnp.zeros((ZC, D), F32)).reshape(NC, NP, D)


def _deg_body(dst_hbm, ones_hbm, zeros_hbm, out_hbm, dbuf, zbuf, ones_v, acc_sp):
    c = lax.axis_index("c")
    s = lax.axis_index("s")
    wid = s * NC + c
    _zero_phase(zeros_hbm, zbuf, acc_sp, s)
    pltpu.sync_copy(ones_hbm, ones_v)
    plsc.subcore_barrier()

    def body(i, carry):
        base = wid * EPW + i * C
        pltpu.sync_copy(dst_hbm.at[pl.ds(base, C)], dbuf)
        pltpu.sync_copy(ones_v, acc_sp.at[dbuf], add=True)
        return carry

    lax.fori_loop(0, EPW // C, body, 0)
    plsc.subcore_barrier()
    _writeback(out_hbm, zbuf, acc_sp, c, s)


def _deg_call(dst):
    f = pl.kernel(
        _deg_body,
        out_type=jax.ShapeDtypeStruct((NC * NP, D), F32),
        mesh=_mesh(),
        scratch_types=[
            pltpu.VMEM((C,), I32),
            pltpu.VMEM((ZC, D), F32),
            pltpu.VMEM((C, D), F32),
            pltpu.VMEM_SHARED((NP, D), F32),
        ],
    )
    return f(dst, jnp.ones((C, D), F32), jnp.zeros((ZC, D), F32)).reshape(NC, NP, D)


SEL_PAD = 32
GPW = B * SEL_PAD // NW


def _selgather_body(h_hbm, idx_hbm, out_hbm, ibuf, rows, sem):
    c = lax.axis_index("c")
    s = lax.axis_index("s")
    wid = s * NC + c
    base = wid * GPW
    pltpu.sync_copy(idx_hbm.at[pl.ds(base, GPW)], ibuf)
    pltpu.async_copy(h_hbm.at[ibuf], rows, sem).wait()
    pltpu.sync_copy(rows, out_hbm.at[pl.ds(base, GPW)])


def _selgather_call(h, idx):
    f = pl.kernel(
        _selgather_body,
        out_type=jax.ShapeDtypeStruct((B * SEL_PAD, D), F32),
        mesh=_mesh(),
        scratch_types=[
            pltpu.VMEM((GPW,), I32),
            pltpu.VMEM((GPW, D), F32),
            pltpu.SemaphoreType.DMA,
        ],
    )
    return f(h, idx)


BLK = 2000


def _layer_kbody(p_ref, d_ref, h_ref, wl_ref, wr_ref, b_ref, o_ref, k_ref):
    p = p_ref[0] + p_ref[1]
    deg = d_ref[0][:, 0:1] + d_ref[1][:, 0:1]
    inv = 1.0 / jnp.maximum(deg, 1.0)
    aggr = p * inv
    h = h_ref[...]
    o = jnp.dot(aggr, wl_ref[...], preferred_element_type=F32)
    o = o + jnp.dot(h, wr_ref[...], preferred_element_type=F32)
    o = jnp.maximum(o + b_ref[...], 0.0)
    o_ref[...] = o
    k_ref[...] = o[:, D - 1:D]


def _layer_call(p, degp, h, Wl, Wr, b):
    grid = N // BLK
    return pl.pallas_call(
        _layer_kbody,
        grid=(grid,),
        in_specs=[
            pl.BlockSpec((NC, BLK, D), lambda i: (0, i, 0)),
            pl.BlockSpec((NC, BLK, D), lambda i: (0, i, 0)),
            pl.BlockSpec((BLK, D), lambda i: (i, 0)),
            pl.BlockSpec((D, D), lambda i: (0, 0)),
            pl.BlockSpec((D, D), lambda i: (0, 0)),
            pl.BlockSpec((1, D), lambda i: (0, 0)),
        ],
        out_specs=[
            pl.BlockSpec((BLK, D), lambda i: (i, 0)),
            pl.BlockSpec((BLK, 1), lambda i: (i, 0)),
        ],
        out_shape=[
            jax.ShapeDtypeStruct((N, D), F32),
            jax.ShapeDtypeStruct((N, 1), F32),
        ],
    )(p, degp, h, Wl, Wr, b.reshape(1, D))


NEG = -3.0e38
BIGI = 2 ** 30


def _select_kbody(key_ref, batch_ref, oidx_ref, oval_ref):
    key = key_ref[...]
    batch = batch_ref[...]
    g = lax.broadcasted_iota(I32, (B, N), 0)
    col = lax.broadcasted_iota(I32, (B, N), 1)
    Dm = jnp.where(batch == g, key, NEG)
    oidx_ref[...] = jnp.zeros((B, SEL_PAD), I32)
    oval_ref[...] = jnp.zeros((B, SEL_PAD), F32)
    for k in range(K):
        m = jnp.max(Dm, axis=1, keepdims=True)
        pos = jnp.min(jnp.where(Dm == m, col, BIGI), axis=1, keepdims=True)
        oidx_ref[:, k:k + 1] = pos
        oval_ref[:, k:k + 1] = jnp.where(m > -1.0e30, 1.0, 0.0)
        Dm = jnp.where(col == pos, NEG, Dm)


def _select_call(key_row, batch_row):
    return pl.pallas_call(
        _select_kbody,
        out_shape=[
            jax.ShapeDtypeStruct((B, SEL_PAD), I32),
            jax.ShapeDtypeStruct((B, SEL_PAD), F32),
        ],
    )(key_row, batch_row)


def _head_kbody(rows_ref, val_ref, wc_ref, cb_ref, w5_ref, b1_ref,
                w2_ref, b2_ref, o_ref):
    rows = rows_ref[...] * val_ref[...][:, :, None]
    h1 = jnp.zeros((B, D), F32)
    for p in range(K - 5 + 1):
        acc = jnp.broadcast_to(cb_ref[...], (B, 32))
        for t in range(5):
            acc = acc + jnp.dot(rows[:, p + t, :], wc_ref[t],
                                preferred_element_type=F32)
        acc = jnp.maximum(acc, 0.0)
        h1 = h1 + jnp.dot(acc, w5_ref[p], preferred_element_type=F32)
    h1 = jnp.maximum(h1 + b1_ref[...], 0.0)
    logits = jnp.dot(h1, w2_ref[...], preferred_element_type=F32) + b2_ref[...]
    m = jnp.max(logits, axis=1, keepdims=True)
    z = logits - m
    o_ref[...] = z - jnp.log(jnp.sum(jnp.exp(z), axis=1, keepdims=True))


def _head_call(rows, valid, conv1d_w, conv1d_b, lin1_w, lin1_b, lin2_w, lin2_b):
    NCLS = lin2_w.shape[1]
    wc = jnp.transpose(conv1d_w, (2, 1, 0))
    w5 = jnp.transpose(lin1_w.reshape(32, K - 4, D), (1, 0, 2))
    return pl.pallas_call(
        _head_kbody,
        out_shape=jax.ShapeDtypeStruct((B, NCLS), F32),
    )(rows, valid, wc, conv1d_b.reshape(1, 32), w5,
      lin1_b.reshape(1, D), lin2_w, lin2_b.reshape(1, NCLS))


def kernel(x, edge_index, batch, W1_l, W1_r, b1, W2_l, W2_r, b2,
           W3_l, W3_r, b3, conv1d_w, conv1d_b, lin1_w, lin1_b,
           lin2_w, lin2_b):
    src = edge_index[0]
    dst = edge_index[1]

    degp = _deg_call(dst)

    p = _aggr_call(x, src, dst)
    h1, _ = _layer_call(p, degp, x, W1_l, W1_r, b1)
    p = _aggr_call(h1, src, dst)
    h2, _ = _layer_call(p, degp, h1, W2_l, W2_r, b2)
    p = _aggr_call(h2, src, dst)
    h3, key = _layer_call(p, degp, h2, W3_l, W3_r, b3)

    sel_idx, valid = _select_call(key.reshape(1, N), batch.reshape(1, N))
    rows = _selgather_call(h3, sel_idx.reshape(-1))
    return _head_call(rows.reshape(B, SEL_PAD, D), valid,
                      conv1d_w, conv1d_b, lin1_w, lin1_b, lin2_w, lin2_b)


# speedup vs baseline: 1.0794x; 1.0794x over previous
import jax
import jax.numpy as jnp
from jax import lax
from jax.experimental import pallas as pl
from jax.experimental.pallas import tpu as pltpu
from jax.experimental.pallas import tpu_sc as plsc

F32 = jnp.float32
I32 = jnp.int32

NC = 2
NS = 16
NW = NC * NS

N = 10000
E = 320000
D = 128
B = 64
K = 30
EPW = E // NW
C = 200
NP = 10240
RPT = NP // NS
ZC = 64

_mesh = lambda: plsc.VectorSubcoreMesh(core_axis_name="c", subcore_axis_name="s")


def _zero_phase(zeros_hbm, zbuf, acc_sp, s):
    pltpu.sync_copy(zeros_hbm, zbuf)
    for k in range(RPT // ZC):
        pltpu.sync_copy(zbuf, acc_sp.at[pl.ds(s * RPT + k * ZC, ZC)])


def _writeback(out_hbm, zbuf, acc_sp, c, s):
    for k in range(RPT // ZC):
        pltpu.sync_copy(acc_sp.at[pl.ds(s * RPT + k * ZC, ZC)], zbuf)
        pltpu.sync_copy(zbuf, out_hbm.at[pl.ds(c * NP + s * RPT + k * ZC, ZC)])


def _aggr_body(h_hbm, src_hbm, dst_hbm, zeros_hbm, out_hbm,
               sbuf, dbuf, zbuf, rows, acc_sp, sem):
    c = lax.axis_index("c")
    s = lax.axis_index("s")
    wid = s * NC + c
    _zero_phase(zeros_hbm, zbuf, acc_sp, s)
    plsc.subcore_barrier()

    def body(i, carry):
        base = wid * EPW + i * C
        pltpu.sync_copy(src_hbm.at[pl.ds(base, C)], sbuf)
        pltpu.sync_copy(dst_hbm.at[pl.ds(base, C)], dbuf)
        pltpu.async_copy(h_hbm.at[sbuf], rows, sem).wait()
        pltpu.sync_copy(rows, acc_sp.at[dbuf], add=True)
        return carry

    lax.fori_loop(0, EPW // C, body, 0)
    plsc.subcore_barrier()
    _writeback(out_hbm, zbuf, acc_sp, c, s)


def _aggr_call(h, src, dst):
    f = pl.kernel(
        _aggr_body,
        out_type=jax.ShapeDtypeStruct((NC * NP, D), F32),
        mesh=_mesh(),
        scratch_types=[
            pltpu.VMEM((C,), I32),
            pltpu.VMEM((C,), I32),
            pltpu.VMEM((ZC, D), F32),
            pltpu.VMEM((C, D), F32),
            pltpu.VMEM_SHARED((NP, D), F32),
            pltpu.SemaphoreType.DMA,
        ],
    )
    return f(h, src, dst, jnp.zeros((ZC, D), F32)).reshape(NC, NP, D)


def _deg_body(dst_hbm, ones_hbm, zeros_hbm, out_hbm, dbuf, zbuf, ones_v, acc_sp):
    c = lax.axis_index("c")
    s = lax.axis_index("s")
    wid = s * NC + c
    _zero_phase(zeros_hbm, zbuf, acc_sp, s)
    pltpu.sync_copy(ones_hbm, ones_v)
    plsc.subcore_barrier()

    def body(i, carry):
        base = wid * EPW + i * C
        pltpu.sync_copy(dst_hbm.at[pl.ds(base, C)], dbuf)
        pltpu.sync_copy(ones_v, acc_sp.at[dbuf], add=True)
        return carry

    lax.fori_loop(0, EPW // C, body, 0)
    plsc.subcore_barrier()
    _writeback(out_hbm, zbuf, acc_sp, c, s)


def _deg_call(dst):
    f = pl.kernel(
        _deg_body,
        out_type=jax.ShapeDtypeStruct((NC * NP, D), F32),
        mesh=_mesh(),
        scratch_types=[
            pltpu.VMEM((C,), I32),
            pltpu.VMEM((ZC, D), F32),
            pltpu.VMEM((C, D), F32),
            pltpu.VMEM_SHARED((NP, D), F32),
        ],
    )
    return f(dst, jnp.ones((C, D), F32), jnp.zeros((ZC, D), F32)).reshape(NC, NP, D)


SEL_PAD = 32
GPW = B * SEL_PAD // NW


def _selgather_body(h_hbm, idx_hbm, out_hbm, ibuf, rows, sem):
    c = lax.axis_index("c")
    s = lax.axis_index("s")
    wid = s * NC + c
    base = wid * GPW
    pltpu.sync_copy(idx_hbm.at[pl.ds(base, GPW)], ibuf)
    pltpu.async_copy(h_hbm.at[ibuf], rows, sem).wait()
    pltpu.sync_copy(rows, out_hbm.at[pl.ds(base, GPW)])


def _selgather_call(h, idx):
    f = pl.kernel(
        _selgather_body,
        out_type=jax.ShapeDtypeStruct((B * SEL_PAD, D), F32),
        mesh=_mesh(),
        scratch_types=[
            pltpu.VMEM((GPW,), I32),
            pltpu.VMEM((GPW, D), F32),
            pltpu.SemaphoreType.DMA,
        ],
    )
    return f(h, idx)


BLK = 2000


def _layer_kbody(p_ref, d_ref, h_ref, wl_ref, wr_ref, b_ref, o_ref, k_ref):
    p = p_ref[0] + p_ref[1]
    deg = d_ref[0][:, 0:1] + d_ref[1][:, 0:1]
    inv = 1.0 / jnp.maximum(deg, 1.0)
    aggr = p * inv
    h = h_ref[...]
    o = jnp.dot(aggr, wl_ref[...], preferred_element_type=F32)
    o = o + jnp.dot(h, wr_ref[...], preferred_element_type=F32)
    o = jnp.maximum(o + b_ref[...], 0.0)
    o_ref[...] = o
    k_ref[...] = o[:, D - 1:D]


def _layer_call(p, degp, h, Wl, Wr, b):
    grid = N // BLK
    return pl.pallas_call(
        _layer_kbody,
        grid=(grid,),
        in_specs=[
            pl.BlockSpec((NC, BLK, D), lambda i: (0, i, 0)),
            pl.BlockSpec((NC, BLK, D), lambda i: (0, i, 0)),
            pl.BlockSpec((BLK, D), lambda i: (i, 0)),
            pl.BlockSpec((D, D), lambda i: (0, 0)),
            pl.BlockSpec((D, D), lambda i: (0, 0)),
            pl.BlockSpec((1, D), lambda i: (0, 0)),
        ],
        out_specs=[
            pl.BlockSpec((BLK, D), lambda i: (i, 0)),
            pl.BlockSpec((BLK, 1), lambda i: (i, 0)),
        ],
        out_shape=[
            jax.ShapeDtypeStruct((N, D), F32),
            jax.ShapeDtypeStruct((N, 1), F32),
        ],
    )(p, degp, h, Wl, Wr, b.reshape(1, D))


NEG = -3.0e38
BIGI = 2 ** 30


def _select_kbody(key_ref, batch_ref, oidx_ref, oval_ref):
    key = key_ref[...]
    batch = batch_ref[...]
    g = lax.broadcasted_iota(I32, (B, N), 0)
    col = lax.broadcasted_iota(I32, (B, N), 1)
    Dm = jnp.where(batch == g, key, NEG)
    oidx_ref[...] = jnp.zeros((B, SEL_PAD), I32)
    oval_ref[...] = jnp.zeros((B, SEL_PAD), F32)
    for k in range(K):
        m = jnp.max(Dm, axis=1, keepdims=True)
        pos = jnp.min(jnp.where(Dm == m, col, BIGI), axis=1, keepdims=True)
        oidx_ref[:, k:k + 1] = pos
        oval_ref[:, k:k + 1] = jnp.where(m > -1.0e30, 1.0, 0.0)
        Dm = jnp.where(col == pos, NEG, Dm)


def _select_call(key_row, batch_row):
    return pl.pallas_call(
        _select_kbody,
        out_shape=[
            jax.ShapeDtypeStruct((B, SEL_PAD), I32),
            jax.ShapeDtypeStruct((B, SEL_PAD), F32),
        ],
    )(key_row, batch_row)


def _head_kbody(rows_ref, val_ref, wc_ref, cb_ref, w5_ref, b1_ref,
                w2_ref, b2_ref, o_ref):
    rows = rows_ref[...] * val_ref[...][:, :, None]
    h1 = jnp.zeros((B, D), F32)
    for p in range(K - 5 + 1):
        acc = jnp.broadcast_to(cb_ref[...], (B, 32))
        for t in range(5):
            acc = acc + jnp.dot(rows[:, p + t, :], wc_ref[t],
                                preferred_element_type=F32)
        acc = jnp.maximum(acc, 0.0)
        h1 = h1 + jnp.dot(acc, w5_ref[p], preferred_element_type=F32)
    h1 = jnp.maximum(h1 + b1_ref[...], 0.0)
    logits = jnp.dot(h1, w2_ref[...], preferred_element_type=F32) + b2_ref[...]
    m = jnp.max(logits, axis=1, keepdims=True)
    z = logits - m
    o_ref[...] = z - jnp.log(jnp.sum(jnp.exp(z), axis=1, keepdims=True))


def _head_call(rows, valid, conv1d_w, conv1d_b, lin1_w, lin1_b, lin2_w, lin2_b):
    NCLS = lin2_w.shape[1]
    wc = jnp.transpose(conv1d_w, (2, 1, 0))
    w5 = jnp.transpose(lin1_w.reshape(32, K - 4, D), (1, 0, 2))
    return pl.pallas_call(
        _head_kbody,
        out_shape=jax.ShapeDtypeStruct((B, NCLS), F32),
    )(rows, valid, wc, conv1d_b.reshape(1, 32), w5,
      lin1_b.reshape(1, D), lin2_w, lin2_b.reshape(1, NCLS))


def kernel(x, edge_index, batch, W1_l, W1_r, b1, W2_l, W2_r, b2,
           W3_l, W3_r, b3, conv1d_w, conv1d_b, lin1_w, lin1_b,
           lin2_w, lin2_b):
    src = edge_index[0]
    dst = edge_index[1]

    degp = _deg_call(dst)

    p = _aggr_call(x, src, dst)
    h1, _ = _layer_call(p, degp, x, W1_l, W1_r, b1)
    p = _aggr_call(h1, src, dst)
    h2, _ = _layer_call(p, degp, h1, W2_l, W2_r, b2)
    p = _aggr_call(h2, src, dst)
    h3, key = _layer_call(p, degp, h2, W3_l, W3_r, b3)

    sel_idx, valid = _select_call(key.reshape(1, N), batch.reshape(1, N))
    rows = _selgather_call(h3, sel_idx.reshape(-1))
    return _head_call(rows.reshape(B, SEL_PAD, D), valid,
                      conv1d_w, conv1d_b, lin1_w, lin1_b, lin2_w, lin2_b)


# speedup vs baseline: 1.2927x; 1.1976x over previous
import jax
import jax.numpy as jnp
from jax import lax
from jax.experimental import pallas as pl
from jax.experimental.pallas import tpu as pltpu
from jax.experimental.pallas import tpu_sc as plsc

F32 = jnp.float32
I32 = jnp.int32

NC = 2
NS = 16
NW = NC * NS

N = 10000
E = 320000
D = 128
B = 64
K = 30
EPW = E // NW
C = 200
NP = 10240
RPT = NP // NS
ZC = 64

_mesh = lambda: plsc.VectorSubcoreMesh(core_axis_name="c", subcore_axis_name="s")


def _zero_phase(zeros_hbm, zbuf, acc_sp, s):
    pltpu.sync_copy(zeros_hbm, zbuf)
    for k in range(RPT // ZC):
        pltpu.sync_copy(zbuf, acc_sp.at[pl.ds(s * RPT + k * ZC, ZC)])


def _writeback(out_hbm, zbuf, acc_sp, c, s):
    for k in range(RPT // ZC):
        pltpu.sync_copy(acc_sp.at[pl.ds(s * RPT + k * ZC, ZC)], zbuf)
        pltpu.sync_copy(zbuf, out_hbm.at[pl.ds(c * NP + s * RPT + k * ZC, ZC)])


def _aggr_body(h_hbm, src_hbm, dst_hbm, zeros_hbm, out_hbm,
               sA, dA, sB, dB, zbuf, rows, acc_sp, semG, semI):
    c = lax.axis_index("c")
    s = lax.axis_index("s")
    wid = s * NC + c
    _zero_phase(zeros_hbm, zbuf, acc_sp, s)
    plsc.subcore_barrier()

    NCH = EPW // C

    def idx_start(i, sb, db):
        base = wid * EPW + i * C
        pltpu.async_copy(src_hbm.at[pl.ds(base, C)], sb, semI)
        pltpu.async_copy(dst_hbm.at[pl.ds(base, C)], db, semI)

    def idx_wait(i, sb, db):
        base = wid * EPW + i * C
        pltpu.make_async_copy(src_hbm.at[pl.ds(base, C)], sb, semI).wait()
        pltpu.make_async_copy(dst_hbm.at[pl.ds(base, C)], db, semI).wait()

    def chunk(sb, db):
        pltpu.async_copy(h_hbm.at[sb], rows, semG).wait()
        pltpu.sync_copy(rows, acc_sp.at[db], add=True)

    idx_start(0, sA, dA)

    def body(i, carry):
        a = 2 * i
        b = 2 * i + 1
        idx_wait(a, sA, dA)
        idx_start(b, sB, dB)
        chunk(sA, dA)
        idx_wait(b, sB, dB)

        @pl.when(b + 1 < NCH)
        def _():
            idx_start(b + 1, sA, dA)

        chunk(sB, dB)
        return carry

    lax.fori_loop(0, NCH // 2, body, 0)
    plsc.subcore_barrier()
    _writeback(out_hbm, zbuf, acc_sp, c, s)


def _aggr_call(h, src, dst):
    f = pl.kernel(
        _aggr_body,
        out_type=jax.ShapeDtypeStruct((NC * NP, D), F32),
        mesh=_mesh(),
        scratch_types=[
            pltpu.VMEM((C,), I32),
            pltpu.VMEM((C,), I32),
            pltpu.VMEM((C,), I32),
            pltpu.VMEM((C,), I32),
            pltpu.VMEM((ZC, D), F32),
            pltpu.VMEM((C, D), F32),
            pltpu.VMEM_SHARED((NP, D), F32),
            pltpu.SemaphoreType.DMA,
            pltpu.SemaphoreType.DMA,
        ],
    )
    return f(h, src, dst, jnp.zeros((ZC, D), F32)).reshape(NC, NP, D)


def _deg_body(dst_hbm, ones_hbm, zeros_hbm, out_hbm, dbuf, zbuf, ones_v, acc_sp):
    c = lax.axis_index("c")
    s = lax.axis_index("s")
    wid = s * NC + c
    _zero_phase(zeros_hbm, zbuf, acc_sp, s)
    pltpu.sync_copy(ones_hbm, ones_v)
    plsc.subcore_barrier()

    def body(i, carry):
        base = wid * EPW + i * C
        pltpu.sync_copy(dst_hbm.at[pl.ds(base, C)], dbuf)
        pltpu.sync_copy(ones_v, acc_sp.at[dbuf], add=True)
        return carry

    lax.fori_loop(0, EPW // C, body, 0)
    plsc.subcore_barrier()
    _writeback(out_hbm, zbuf, acc_sp, c, s)


def _deg_call(dst):
    f = pl.kernel(
        _deg_body,
        out_type=jax.ShapeDtypeStruct((NC * NP, D), F32),
        mesh=_mesh(),
        scratch_types=[
            pltpu.VMEM((C,), I32),
            pltpu.VMEM((ZC, D), F32),
            pltpu.VMEM((C, D), F32),
            pltpu.VMEM_SHARED((NP, D), F32),
        ],
    )
    return f(dst, jnp.ones((C, D), F32), jnp.zeros((ZC, D), F32)).reshape(NC, NP, D)


SEL_PAD = 32
GPW = B * SEL_PAD // NW


def _selgather_body(h_hbm, idx_hbm, out_hbm, ibuf, rows, sem):
    c = lax.axis_index("c")
    s = lax.axis_index("s")
    wid = s * NC + c
    base = wid * GPW
    pltpu.sync_copy(idx_hbm.at[pl.ds(base, GPW)], ibuf)
    pltpu.async_copy(h_hbm.at[ibuf], rows, sem).wait()
    pltpu.sync_copy(rows, out_hbm.at[pl.ds(base, GPW)])


def _selgather_call(h, idx):
    f = pl.kernel(
        _selgather_body,
        out_type=jax.ShapeDtypeStruct((B * SEL_PAD, D), F32),
        mesh=_mesh(),
        scratch_types=[
            pltpu.VMEM((GPW,), I32),
            pltpu.VMEM((GPW, D), F32),
            pltpu.SemaphoreType.DMA,
        ],
    )
    return f(h, idx)


BLK = 2000


def _layer_kbody(p_ref, d_ref, h_ref, wl_ref, wr_ref, b_ref, o_ref, k_ref):
    p = p_ref[0] + p_ref[1]
    deg = d_ref[0][:, 0:1] + d_ref[1][:, 0:1]
    inv = 1.0 / jnp.maximum(deg, 1.0)
    aggr = p * inv
    h = h_ref[...]
    o = jnp.dot(aggr, wl_ref[...], preferred_element_type=F32)
    o = o + jnp.dot(h, wr_ref[...], preferred_element_type=F32)
    o = jnp.maximum(o + b_ref[...], 0.0)
    o_ref[...] = o
    k_ref[...] = o[:, D - 1:D]


def _layer_call(p, degp, h, Wl, Wr, b):
    grid = N // BLK
    return pl.pallas_call(
        _layer_kbody,
        grid=(grid,),
        in_specs=[
            pl.BlockSpec((NC, BLK, D), lambda i: (0, i, 0)),
            pl.BlockSpec((NC, BLK, D), lambda i: (0, i, 0)),
            pl.BlockSpec((BLK, D), lambda i: (i, 0)),
            pl.BlockSpec((D, D), lambda i: (0, 0)),
            pl.BlockSpec((D, D), lambda i: (0, 0)),
            pl.BlockSpec((1, D), lambda i: (0, 0)),
        ],
        out_specs=[
            pl.BlockSpec((BLK, D), lambda i: (i, 0)),
            pl.BlockSpec((BLK, 1), lambda i: (i, 0)),
        ],
        out_shape=[
            jax.ShapeDtypeStruct((N, D), F32),
            jax.ShapeDtypeStruct((N, 1), F32),
        ],
    )(p, degp, h, Wl, Wr, b.reshape(1, D))


NEG = -3.0e38
BIGI = 2 ** 30


def _select_kbody(key_ref, batch_ref, oidx_ref, oval_ref):
    key = key_ref[...]
    batch = batch_ref[...]
    g = lax.broadcasted_iota(I32, (B, N), 0)
    col = lax.broadcasted_iota(I32, (B, N), 1)
    Dm = jnp.where(batch == g, key, NEG)
    oidx_ref[...] = jnp.zeros((B, SEL_PAD), I32)
    oval_ref[...] = jnp.zeros((B, SEL_PAD), F32)
    for k in range(K):
        m = jnp.max(Dm, axis=1, keepdims=True)
        pos = jnp.min(jnp.where(Dm == m, col, BIGI), axis=1, keepdims=True)
        oidx_ref[:, k:k + 1] = pos
        oval_ref[:, k:k + 1] = jnp.where(m > -1.0e30, 1.0, 0.0)
        Dm = jnp.where(col == pos, NEG, Dm)


def _select_call(key_row, batch_row):
    return pl.pallas_call(
        _select_kbody,
        out_shape=[
            jax.ShapeDtypeStruct((B, SEL_PAD), I32),
            jax.ShapeDtypeStruct((B, SEL_PAD), F32),
        ],
    )(key_row, batch_row)


def _head_kbody(rows_ref, val_ref, wc_ref, cb_ref, w5_ref, b1_ref,
                w2_ref, b2_ref, o_ref):
    rows = rows_ref[...] * val_ref[...][:, :, None]
    h1 = jnp.zeros((B, D), F32)
    for p in range(K - 5 + 1):
        acc = jnp.broadcast_to(cb_ref[...], (B, 32))
        for t in range(5):
            acc = acc + jnp.dot(rows[:, p + t, :], wc_ref[t],
                                preferred_element_type=F32)
        acc = jnp.maximum(acc, 0.0)
        h1 = h1 + jnp.dot(acc, w5_ref[p], preferred_element_type=F32)
    h1 = jnp.maximum(h1 + b1_ref[...], 0.0)
    logits = jnp.dot(h1, w2_ref[...], preferred_element_type=F32) + b2_ref[...]
    m = jnp.max(logits, axis=1, keepdims=True)
    z = logits - m
    o_ref[...] = z - jnp.log(jnp.sum(jnp.exp(z), axis=1, keepdims=True))


def _head_call(rows, valid, conv1d_w, conv1d_b, lin1_w, lin1_b, lin2_w, lin2_b):
    NCLS = lin2_w.shape[1]
    wc = jnp.transpose(conv1d_w, (2, 1, 0))
    w5 = jnp.transpose(lin1_w.reshape(32, K - 4, D), (1, 0, 2))
    return pl.pallas_call(
        _head_kbody,
        out_shape=jax.ShapeDtypeStruct((B, NCLS), F32),
    )(rows, valid, wc, conv1d_b.reshape(1, 32), w5,
      lin1_b.reshape(1, D), lin2_w, lin2_b.reshape(1, NCLS))


def kernel(x, edge_index, batch, W1_l, W1_r, b1, W2_l, W2_r, b2,
           W3_l, W3_r, b3, conv1d_w, conv1d_b, lin1_w, lin1_b,
           lin2_w, lin2_b):
    src = edge_index[0]
    dst = edge_index[1]

    degp = _deg_call(dst)

    p = _aggr_call(x, src, dst)
    h1, _ = _layer_call(p, degp, x, W1_l, W1_r, b1)
    p = _aggr_call(h1, src, dst)
    h2, _ = _layer_call(p, degp, h1, W2_l, W2_r, b2)
    p = _aggr_call(h2, src, dst)
    h3, key = _layer_call(p, degp, h2, W3_l, W3_r, b3)

    sel_idx, valid = _select_call(key.reshape(1, N), batch.reshape(1, N))
    rows = _selgather_call(h3, sel_idx.reshape(-1))
    return _head_call(rows.reshape(B, SEL_PAD, D), valid,
                      conv1d_w, conv1d_b, lin1_w, lin1_b, lin2_w, lin2_b)


# speedup vs baseline: 1.3279x; 1.0272x over previous
import jax
import jax.numpy as jnp
from jax import lax
from jax.experimental import pallas as pl
from jax.experimental.pallas import tpu as pltpu
from jax.experimental.pallas import tpu_sc as plsc

F32 = jnp.float32
I32 = jnp.int32

NC = 2
NS = 16
NW = NC * NS

N = 10000
E = 320000
D = 128
B = 64
K = 30
EPW = E // NW
C = 200
NP = 10240
RPT = NP // NS
ZC = 64

_mesh = lambda: plsc.VectorSubcoreMesh(core_axis_name="c", subcore_axis_name="s")


def _zero_phase(zeros_hbm, zbuf, acc_sp, s):
    pltpu.sync_copy(zeros_hbm, zbuf)
    for k in range(RPT // ZC):
        pltpu.sync_copy(zbuf, acc_sp.at[pl.ds(s * RPT + k * ZC, ZC)])


def _writeback(out_hbm, zbuf, acc_sp, c, s):
    for k in range(RPT // ZC):
        pltpu.sync_copy(acc_sp.at[pl.ds(s * RPT + k * ZC, ZC)], zbuf)
        pltpu.sync_copy(zbuf, out_hbm.at[pl.ds(c * NP + s * RPT + k * ZC, ZC)])


def _aggr_body(h_hbm, src_hbm, dst_hbm, zeros_hbm, out_hbm,
               sA, dA, sB, dB, zbuf, rows, acc_sp, semG, semI):
    c = lax.axis_index("c")
    s = lax.axis_index("s")
    wid = s * NC + c
    _zero_phase(zeros_hbm, zbuf, acc_sp, s)
    plsc.subcore_barrier()

    NCH = EPW // C

    def idx_start(i, sb, db):
        base = wid * EPW + i * C
        pltpu.async_copy(src_hbm.at[pl.ds(base, C)], sb, semI)
        pltpu.async_copy(dst_hbm.at[pl.ds(base, C)], db, semI)

    def idx_wait(i, sb, db):
        base = wid * EPW + i * C
        pltpu.make_async_copy(src_hbm.at[pl.ds(base, C)], sb, semI).wait()
        pltpu.make_async_copy(dst_hbm.at[pl.ds(base, C)], db, semI).wait()

    def chunk(sb, db):
        pltpu.async_copy(h_hbm.at[sb], rows, semG).wait()
        pltpu.sync_copy(rows, acc_sp.at[db], add=True)

    idx_start(0, sA, dA)

    def body(i, carry):
        a = 2 * i
        b = 2 * i + 1
        idx_wait(a, sA, dA)
        idx_start(b, sB, dB)
        chunk(sA, dA)
        idx_wait(b, sB, dB)

        @pl.when(b + 1 < NCH)
        def _():
            idx_start(b + 1, sA, dA)

        chunk(sB, dB)
        return carry

    lax.fori_loop(0, NCH // 2, body, 0)
    plsc.subcore_barrier()
    _writeback(out_hbm, zbuf, acc_sp, c, s)


def _aggr_call(h, src, dst):
    f = pl.kernel(
        _aggr_body,
        out_type=jax.ShapeDtypeStruct((NC * NP, D), F32),
        mesh=_mesh(),
        scratch_types=[
            pltpu.VMEM((C,), I32),
            pltpu.VMEM((C,), I32),
            pltpu.VMEM((C,), I32),
            pltpu.VMEM((C,), I32),
            pltpu.VMEM((ZC, D), F32),
            pltpu.VMEM((C, D), F32),
            pltpu.VMEM_SHARED((NP, D), F32),
            pltpu.SemaphoreType.DMA,
            pltpu.SemaphoreType.DMA,
        ],
    )
    return f(h, src, dst, jnp.zeros((ZC, D), F32)).reshape(NC, NP, D)


def _deg_body(dst_hbm, ones_hbm, zeros_hbm, out_hbm, dA, dB, zbuf, ones_v,
              acc_sp, semI):
    c = lax.axis_index("c")
    s = lax.axis_index("s")
    wid = s * NC + c
    _zero_phase(zeros_hbm, zbuf, acc_sp, s)
    pltpu.sync_copy(ones_hbm, ones_v)
    plsc.subcore_barrier()

    NCH = EPW // C

    def idx_start(i, db):
        pltpu.async_copy(dst_hbm.at[pl.ds(wid * EPW + i * C, C)], db, semI)

    def idx_wait(i, db):
        pltpu.make_async_copy(dst_hbm.at[pl.ds(wid * EPW + i * C, C)], db,
                              semI).wait()

    idx_start(0, dA)

    def body(i, carry):
        a = 2 * i
        b = 2 * i + 1
        idx_wait(a, dA)
        idx_start(b, dB)
        pltpu.sync_copy(ones_v, acc_sp.at[dA], add=True)
        idx_wait(b, dB)

        @pl.when(b + 1 < NCH)
        def _():
            idx_start(b + 1, dA)

        pltpu.sync_copy(ones_v, acc_sp.at[dB], add=True)
        return carry

    lax.fori_loop(0, NCH // 2, body, 0)
    plsc.subcore_barrier()
    _writeback(out_hbm, zbuf, acc_sp, c, s)


def _deg_call(dst):
    f = pl.kernel(
        _deg_body,
        out_type=jax.ShapeDtypeStruct((NC * NP, D), F32),
        mesh=_mesh(),
        scratch_types=[
            pltpu.VMEM((C,), I32),
            pltpu.VMEM((C,), I32),
            pltpu.VMEM((ZC, D), F32),
            pltpu.VMEM((C, D), F32),
            pltpu.VMEM_SHARED((NP, D), F32),
            pltpu.SemaphoreType.DMA,
        ],
    )
    return f(dst, jnp.ones((C, D), F32), jnp.zeros((ZC, D), F32)).reshape(NC, NP, D)


SEL_PAD = 32
GPW = B * SEL_PAD // NW


def _selgather_body(h_hbm, idx_hbm, out_hbm, ibuf, rows, sem):
    c = lax.axis_index("c")
    s = lax.axis_index("s")
    wid = s * NC + c
    base = wid * GPW
    pltpu.sync_copy(idx_hbm.at[pl.ds(base, GPW)], ibuf)
    pltpu.async_copy(h_hbm.at[ibuf], rows, sem).wait()
    pltpu.sync_copy(rows, out_hbm.at[pl.ds(base, GPW)])


def _selgather_call(h, idx):
    f = pl.kernel(
        _selgather_body,
        out_type=jax.ShapeDtypeStruct((B * SEL_PAD, D), F32),
        mesh=_mesh(),
        scratch_types=[
            pltpu.VMEM((GPW,), I32),
            pltpu.VMEM((GPW, D), F32),
            pltpu.SemaphoreType.DMA,
        ],
    )
    return f(h, idx)


BLK = 2000


def _layer_kbody(p_ref, d_ref, h_ref, wl_ref, wr_ref, b_ref, o_ref, k_ref):
    p = p_ref[0] + p_ref[1]
    deg = d_ref[0][:, 0:1] + d_ref[1][:, 0:1]
    inv = 1.0 / jnp.maximum(deg, 1.0)
    aggr = p * inv
    h = h_ref[...]
    o = jnp.dot(aggr, wl_ref[...], preferred_element_type=F32)
    o = o + jnp.dot(h, wr_ref[...], preferred_element_type=F32)
    o = jnp.maximum(o + b_ref[...], 0.0)
    o_ref[...] = o
    k_ref[...] = o[:, D - 1:D]


def _layer_call(p, degp, h, Wl, Wr, b):
    grid = N // BLK
    return pl.pallas_call(
        _layer_kbody,
        grid=(grid,),
        in_specs=[
            pl.BlockSpec((NC, BLK, D), lambda i: (0, i, 0)),
            pl.BlockSpec((NC, BLK, D), lambda i: (0, i, 0)),
            pl.BlockSpec((BLK, D), lambda i: (i, 0)),
            pl.BlockSpec((D, D), lambda i: (0, 0)),
            pl.BlockSpec((D, D), lambda i: (0, 0)),
            pl.BlockSpec((1, D), lambda i: (0, 0)),
        ],
        out_specs=[
            pl.BlockSpec((BLK, D), lambda i: (i, 0)),
            pl.BlockSpec((BLK, 1), lambda i: (i, 0)),
        ],
        out_shape=[
            jax.ShapeDtypeStruct((N, D), F32),
            jax.ShapeDtypeStruct((N, 1), F32),
        ],
    )(p, degp, h, Wl, Wr, b.reshape(1, D))


NEG = -3.0e38
BIGI = 2 ** 30


def _select_kbody(key_ref, batch_ref, oidx_ref, oval_ref):
    key = key_ref[...]
    batch = batch_ref[...]
    g = lax.broadcasted_iota(I32, (B, N), 0)
    col = lax.broadcasted_iota(I32, (B, N), 1)
    Dm = jnp.where(batch == g, key, NEG)
    oidx_ref[...] = jnp.zeros((B, SEL_PAD), I32)
    oval_ref[...] = jnp.zeros((B, SEL_PAD), F32)
    for k in range(K):
        m = jnp.max(Dm, axis=1, keepdims=True)
        pos = jnp.min(jnp.where(Dm == m, col, BIGI), axis=1, keepdims=True)
        oidx_ref[:, k:k + 1] = pos
        oval_ref[:, k:k + 1] = jnp.where(m > -1.0e30, 1.0, 0.0)
        Dm = jnp.where(col == pos, NEG, Dm)


def _select_call(key_row, batch_row):
    return pl.pallas_call(
        _select_kbody,
        out_shape=[
            jax.ShapeDtypeStruct((B, SEL_PAD), I32),
            jax.ShapeDtypeStruct((B, SEL_PAD), F32),
        ],
    )(key_row, batch_row)


def _head_kbody(rows_ref, val_ref, wc_ref, cb_ref, w5_ref, b1_ref,
                w2_ref, b2_ref, o_ref):
    rows = rows_ref[...] * val_ref[...][:, :, None]
    h1 = jnp.zeros((B, D), F32)
    for p in range(K - 5 + 1):
        acc = jnp.broadcast_to(cb_ref[...], (B, 32))
        for t in range(5):
            acc = acc + jnp.dot(rows[:, p + t, :], wc_ref[t],
                                preferred_element_type=F32)
        acc = jnp.maximum(acc, 0.0)
        h1 = h1 + jnp.dot(acc, w5_ref[p], preferred_element_type=F32)
    h1 = jnp.maximum(h1 + b1_ref[...], 0.0)
    logits = jnp.dot(h1, w2_ref[...], preferred_element_type=F32) + b2_ref[...]
    m = jnp.max(logits, axis=1, keepdims=True)
    z = logits - m
    o_ref[...] = z - jnp.log(jnp.sum(jnp.exp(z), axis=1, keepdims=True))


def _head_call(rows, valid, conv1d_w, conv1d_b, lin1_w, lin1_b, lin2_w, lin2_b):
    NCLS = lin2_w.shape[1]
    wc = jnp.transpose(conv1d_w, (2, 1, 0))
    w5 = jnp.transpose(lin1_w.reshape(32, K - 4, D), (1, 0, 2))
    return pl.pallas_call(
        _head_kbody,
        out_shape=jax.ShapeDtypeStruct((B, NCLS), F32),
    )(rows, valid, wc, conv1d_b.reshape(1, 32), w5,
      lin1_b.reshape(1, D), lin2_w, lin2_b.reshape(1, NCLS))


def kernel(x, edge_index, batch, W1_l, W1_r, b1, W2_l, W2_r, b2,
           W3_l, W3_r, b3, conv1d_w, conv1d_b, lin1_w, lin1_b,
           lin2_w, lin2_b):
    src = edge_index[0]
    dst = edge_index[1]

    degp = _deg_call(dst)

    p = _aggr_call(x, src, dst)
    h1, _ = _layer_call(p, degp, x, W1_l, W1_r, b1)
    p = _aggr_call(h1, src, dst)
    h2, _ = _layer_call(p, degp, h1, W2_l, W2_r, b2)
    p = _aggr_call(h2, src, dst)
    h3, key = _layer_call(p, degp, h2, W3_l, W3_r, b3)

    sel_idx, valid = _select_call(key.reshape(1, N), batch.reshape(1, N))
    rows = _selgather_call(h3, sel_idx.reshape(-1))
    return _head_call(rows.reshape(B, SEL_PAD, D), valid,
                      conv1d_w, conv1d_b, lin1_w, lin1_b, lin2_w, lin2_b)
